# baseline (speedup 1.0000x reference)
import functools

import jax
import jax.numpy as jnp
from jax import lax
from jax.experimental import pallas as pl
from jax.experimental.pallas import tpu as pltpu
from jax.experimental.pallas import tpu_sc as plsc

E = 320000
D = 128
G = 256
NC = 2
NS = 16
L = 16
NW = NC * NS
EW = E // NW
C = 80
NCHUNK = EW // C
GROUPS = C // L


def _tanh(z):
    a = jnp.abs(z)
    e = jnp.exp(a * (-2.0))
    return jnp.sign(z) * (1.0 - e) / (1.0 + e)


def _sc_body(feats_hbm, ids_hbm, w_hbm, b_hbm, out_hbm,
             bufA, idsA, bufB, idsB, acc, wbuf, bbuf, semA, semB):
    wid = lax.axis_index("s") * NC + lax.axis_index("c")
    row0 = wid * EW

    pltpu.sync_copy(w_hbm, wbuf)
    pltpu.sync_copy(b_hbm, bbuf)
    bv = bbuf[...]
    wv = [wbuf[pl.ds(j * L, L)] for j in range(D // L)]
    lanes = lax.iota(jnp.int32, L)

    def _zero(i, _):
        acc[pl.ds(i * L, L)] = jnp.zeros((L,), jnp.float32)
        return 0
    lax.fori_loop(0, G * D // L, _zero, 0)

    def _start(ci, buf, ids, sem):
        base = row0 + ci * C
        pltpu.async_copy(feats_hbm.at[pl.ds(base, C)], buf, sem)
        pltpu.async_copy(ids_hbm.at[pl.ds(base, C)], ids, sem)

    def _wait(buf, ids, sem):
        pltpu.make_async_copy(feats_hbm.at[pl.ds(0, C)], buf, sem).wait()
        pltpu.make_async_copy(ids_hbm.at[pl.ds(0, C)], ids, sem).wait()

    def _dg(x, perm):
        return x.at[perm].get(mode="promise_in_bounds")

    perms = [lanes ^ k for k in (1, 2, 4, 8)]
    masks = [(lanes & k) == 0 for k in (1, 2, 4, 8)]

    def _process(buf, ids_v):
        def _group(gi, _):
            r0 = gi * L
            svecs = []
            for r in range(L):
                row = r0 + r
                sa = buf[row, pl.ds(0, L)] * wv[0]
                sb = buf[row, pl.ds(L, L)] * wv[1]
                for j in range(2, D // L, 2):
                    sa = sa + buf[row, pl.ds(j * L, L)] * wv[j]
                    sb = sb + buf[row, pl.ds((j + 1) * L, L)] * wv[j + 1]
                svecs.append(sa + sb)
            v = svecs
            for m, p in zip(masks, perms):
                v = [jnp.where(m, a, b) + jnp.where(m, _dg(a, p), _dg(b, p))
                     for a, b in zip(v[0::2], v[1::2])]
            hv = v[0]
            wg = _tanh(hv + bv)
            idv = jnp.minimum(ids_v[pl.ds(r0, L)], G - 1)
            for r in range(L):
                row = r0 + r
                rfull = jnp.full((L,), r, jnp.int32)
                wsp = wg.at[rfull].get(mode="promise_in_bounds")
                ssp = idv.at[rfull].get(mode="promise_in_bounds")
                base_idx = ssp * D + lanes
                for j in range(D // L):
                    x = buf[row, pl.ds(j * L, L)] * wsp
                    plsc.addupdate_scatter(acc, [base_idx + j * L], x)
            return 0

        lax.fori_loop(0, GROUPS, _group, 0, unroll=2)

    _start(0, bufA, idsA, semA)

    def _pair(p, _):
        ci = p * 2
        _wait(bufA, idsA, semA)
        _start(ci + 1, bufB, idsB, semB)
        _process(bufA, idsA)
        _wait(bufB, idsB, semB)
        _start(ci + 2, bufA, idsA, semA)
        _process(bufB, idsB)
        return 0

    lax.fori_loop(0, NCHUNK // 2, _pair, 0)
    _wait(bufA, idsA, semA)
    _process(bufA, idsA)
    pltpu.sync_copy(acc, out_hbm.at[wid])


def _combine_body(parts_ref, o_ref):
    o_ref[...] = jnp.sum(parts_ref[...], axis=0)


@jax.jit
def _run(edge_feats, ids32, w_flat, b_pad):
    mesh = plsc.VectorSubcoreMesh(core_axis_name="c", subcore_axis_name="s",
                                  num_cores=NC, num_subcores=NS)
    sc = pl.kernel(
        _sc_body,
        out_type=jax.ShapeDtypeStruct((NW, G * D), jnp.float32),
        mesh=mesh,
        compiler_params=pltpu.CompilerParams(needs_layout_passes=False),
        scratch_types=[
            pltpu.VMEM((C, D), jnp.float32),
            pltpu.VMEM((C,), jnp.int32),
            pltpu.VMEM((C, D), jnp.float32),
            pltpu.VMEM((C,), jnp.int32),
            pltpu.VMEM((G * D,), jnp.float32),
            pltpu.VMEM((D,), jnp.float32),
            pltpu.VMEM((L,), jnp.float32),
            pltpu.SemaphoreType.DMA,
            pltpu.SemaphoreType.DMA,
        ],
    )
    parts = sc(edge_feats, ids32, w_flat, b_pad)
    out = pl.pallas_call(
        _combine_body,
        out_shape=jax.ShapeDtypeStruct((G, D), jnp.float32),
    )(parts.reshape(NW, G, D))
    return out


def kernel(edge_feats, segment_ids, num_graphs, W, b):
    ids32 = segment_ids.astype(jnp.int32)
    w_flat = W.reshape(D)
    b_pad = jnp.broadcast_to(b.reshape(1), (L,)).astype(jnp.float32)
    return _run(edge_feats, ids32, w_flat, b_pad)


# speedup vs baseline: 1.7685x; 1.7685x over previous
import functools

import jax
import jax.numpy as jnp
from jax import lax
from jax.experimental import pallas as pl
from jax.experimental.pallas import tpu as pltpu
from jax.experimental.pallas import tpu_sc as plsc

E = 320000
D = 128
G = 256
NC = 2
NS = 16
L = 16
NW = NC * NS
EW = E // NW
C = 80
NCHUNK = EW // C
GROUPS = C // L


def _tanh(z):
    a = jnp.abs(z)
    e = jnp.exp(a * (-2.0))
    return jnp.sign(z) * (1.0 - e) / (1.0 + e)


def _sc_body(feats_hbm, ids_hbm, w_hbm, b_hbm, out_hbm,
             bufA, idsA, bufB, idsB, acc, wbuf, bbuf, semA, semB):
    wid = lax.axis_index("s") * NC + lax.axis_index("c")
    row0 = wid * EW

    pltpu.sync_copy(w_hbm, wbuf)
    pltpu.sync_copy(b_hbm, bbuf)
    bv = bbuf[...]
    wv = [wbuf[pl.ds(j * L, L)] for j in range(D // L)]
    lanes = lax.iota(jnp.int32, L)

    def _zero(i, _):
        acc[pl.ds(i * L, L)] = jnp.zeros((L,), jnp.float32)
        return 0
    lax.fori_loop(0, G * D // L, _zero, 0)

    def _start(ci, buf, ids, sem):
        base = row0 + ci * C
        pltpu.async_copy(feats_hbm.at[pl.ds(base, C)], buf, sem)
        pltpu.async_copy(ids_hbm.at[pl.ds(base, C)], ids, sem)

    def _wait(buf, ids, sem):
        pltpu.make_async_copy(feats_hbm.at[pl.ds(0, C)], buf, sem).wait()
        pltpu.make_async_copy(ids_hbm.at[pl.ds(0, C)], ids, sem).wait()

    def _dg(x, perm):
        return x.at[perm].get(mode="promise_in_bounds")

    perms = [lanes ^ k for k in (1, 2, 4, 8)]
    masks = [(lanes & k) == 0 for k in (1, 2, 4, 8)]

    def _process(buf, ids_v):
        @plsc.parallel_loop(0, GROUPS, unroll=2)
        def _group(gi):
            r0 = gi * L
            svecs = []
            for r in range(L):
                row = r0 + r
                sa = buf[row, pl.ds(0, L)] * wv[0]
                sb = buf[row, pl.ds(L, L)] * wv[1]
                for j in range(2, D // L, 2):
                    sa = sa + buf[row, pl.ds(j * L, L)] * wv[j]
                    sb = sb + buf[row, pl.ds((j + 1) * L, L)] * wv[j + 1]
                svecs.append(sa + sb)
            v = svecs
            for m, p in zip(masks, perms):
                v = [jnp.where(m, a, b) + jnp.where(m, _dg(a, p), _dg(b, p))
                     for a, b in zip(v[0::2], v[1::2])]
            hv = v[0]
            wg = _tanh(hv + bv)
            idv = jnp.minimum(ids_v[pl.ds(r0, L)], G - 1)
            for h in range(2):
                rows = range(8 * h, 8 * h + 8)
                wsps, bases = [], []
                for r in rows:
                    rfull = jnp.full((L,), r, jnp.int32)
                    wsps.append(_dg(wg, rfull))
                    bases.append(_dg(idv, rfull) * D + lanes)
                for j in range(D // L):
                    av = acc.at[pl.ds(j * L, G * D - j * L)]
                    for k, r in enumerate(rows):
                        x = buf[r0 + r, pl.ds(j * L, L)] * wsps[k]
                        plsc.addupdate_scatter(av, [bases[k]], x)

    _start(0, bufA, idsA, semA)

    def _pair(p, _):
        ci = p * 2
        _wait(bufA, idsA, semA)
        _start(ci + 1, bufB, idsB, semB)
        _process(bufA, idsA)
        _wait(bufB, idsB, semB)
        _start(ci + 2, bufA, idsA, semA)
        _process(bufB, idsB)
        return 0

    lax.fori_loop(0, NCHUNK // 2, _pair, 0)
    _wait(bufA, idsA, semA)
    _process(bufA, idsA)
    pltpu.sync_copy(acc, out_hbm.at[wid])


def _combine_body(parts_ref, o_ref):
    o_ref[...] = jnp.sum(parts_ref[...], axis=0)


@jax.jit
def _run(edge_feats, ids32, w_flat, b_pad):
    mesh = plsc.VectorSubcoreMesh(core_axis_name="c", subcore_axis_name="s",
                                  num_cores=NC, num_subcores=NS)
    sc = pl.kernel(
        _sc_body,
        out_type=jax.ShapeDtypeStruct((NW, G * D), jnp.float32),
        mesh=mesh,
        compiler_params=pltpu.CompilerParams(needs_layout_passes=False),
        scratch_types=[
            pltpu.VMEM((C, D), jnp.float32),
            pltpu.VMEM((C,), jnp.int32),
            pltpu.VMEM((C, D), jnp.float32),
            pltpu.VMEM((C,), jnp.int32),
            pltpu.VMEM((G * D,), jnp.float32),
            pltpu.VMEM((D,), jnp.float32),
            pltpu.VMEM((L,), jnp.float32),
            pltpu.SemaphoreType.DMA,
            pltpu.SemaphoreType.DMA,
        ],
    )
    parts = sc(edge_feats, ids32, w_flat, b_pad)
    out = pl.pallas_call(
        _combine_body,
        out_shape=jax.ShapeDtypeStruct((G, D), jnp.float32),
    )(parts.reshape(NW, G, D))
    return out


def kernel(edge_feats, segment_ids, num_graphs, W, b):
    ids32 = segment_ids.astype(jnp.int32)
    w_flat = W.reshape(D)
    b_pad = jnp.broadcast_to(b.reshape(1), (L,)).astype(jnp.float32)
    return _run(edge_feats, ids32, w_flat, b_pad)


# speedup vs baseline: 3.4536x; 1.9528x over previous
import functools

import jax
import jax.numpy as jnp
from jax import lax
from jax.experimental import pallas as pl
from jax.experimental.pallas import tpu as pltpu
from jax.experimental.pallas import tpu_sc as plsc

E = 320000
D = 128
G = 256
NC = 2
NS = 16
L = 16
NW = NC * NS
EW = E // NW
C = 80
NCHUNK = EW // C
GROUPS = C // L


def _tanh(z):
    a = jnp.abs(z)
    e = jnp.exp(a * (-2.0))
    return jnp.sign(z) * (1.0 - e) / (1.0 + e)


def _sc_body(feats_hbm, ids_hbm, w_hbm, b_hbm, out_hbm,
             bufA, idsA, bufB, idsB, acc, wbuf, bbuf, smat, semA, semB):
    wid = lax.axis_index("s") * NC + lax.axis_index("c")
    row0 = wid * EW

    pltpu.sync_copy(w_hbm, wbuf)
    pltpu.sync_copy(b_hbm, bbuf)
    bv = bbuf[...]
    wv = [wbuf[pl.ds(j * L, L)] for j in range(D // L)]
    lanes = lax.iota(jnp.int32, L)

    def _zero(i, _):
        acc[pl.ds(i * L, L)] = jnp.zeros((L,), jnp.float32)
        return 0
    lax.fori_loop(0, G * D // L, _zero, 0)

    def _start(ci, buf, ids, sem):
        base = row0 + ci * C
        pltpu.async_copy(feats_hbm.at[pl.ds(base, C)], buf, sem)
        pltpu.async_copy(ids_hbm.at[pl.ds(base, C)], ids, sem)

    def _wait(buf, ids, sem):
        pltpu.make_async_copy(feats_hbm.at[pl.ds(0, C)], buf, sem).wait()
        pltpu.make_async_copy(ids_hbm.at[pl.ds(0, C)], ids, sem).wait()

    def _dg(x, perm):
        return x.at[perm].get(mode="promise_in_bounds")

    def _process(buf, ids_v):
        def _group(gi, _):
            r0 = gi * L
            for r in range(L):
                row = r0 + r
                s = buf[row, pl.ds(0, L)] * wv[0]
                for j in range(1, D // L):
                    s = s + buf[row, pl.ds(j * L, L)] * wv[j]
                smat[pl.ds(r * L, L)] = s
            hv = plsc.load_gather(smat, [lanes * L])
            for j in range(1, L):
                hv = hv + plsc.load_gather(smat, [lanes * L + j])
            wg = _tanh(hv + bv)
            idv = jnp.minimum(ids_v[pl.ds(r0, L)], G - 1)
            seg_first = idv[0]
            seg_last = idv[L - 1]

            def _uniform(_):
                gsum = [jnp.zeros((L,), jnp.float32) for _ in range(D // L)]
                for r in range(L):
                    wsp = _dg(wg, jnp.full((L,), r, jnp.int32))
                    for j in range(D // L):
                        gsum[j] = gsum[j] + buf[r0 + r, pl.ds(j * L, L)] * wsp
                off = seg_first * D
                for j in range(D // L):
                    plsc.addupdate(acc.at[pl.ds(off + j * L, L)], gsum[j])
                return 0

            def _mixed(_):
                for r in range(L):
                    rfull = jnp.full((L,), r, jnp.int32)
                    wsp = _dg(wg, rfull)
                    base_idx = _dg(idv, rfull) * D + lanes
                    for j in range(D // L):
                        x = buf[r0 + r, pl.ds(j * L, L)] * wsp
                        plsc.addupdate_scatter(acc, [base_idx + j * L], x)
                return 0

            lax.cond(seg_first == seg_last, _uniform, _mixed, 0)
            return 0

        lax.fori_loop(0, GROUPS, _group, 0)

    _start(0, bufA, idsA, semA)

    def _pair(p, _):
        ci = p * 2
        _wait(bufA, idsA, semA)
        _start(ci + 1, bufB, idsB, semB)
        _process(bufA, idsA)
        _wait(bufB, idsB, semB)
        _start(ci + 2, bufA, idsA, semA)
        _process(bufB, idsB)
        return 0

    lax.fori_loop(0, NCHUNK // 2, _pair, 0)
    _wait(bufA, idsA, semA)
    _process(bufA, idsA)
    pltpu.sync_copy(acc, out_hbm.at[wid])


def _combine_body(parts_ref, o_ref):
    o_ref[...] = jnp.sum(parts_ref[...], axis=0)


@jax.jit
def _run(edge_feats, ids32, w_flat, b_pad):
    mesh = plsc.VectorSubcoreMesh(core_axis_name="c", subcore_axis_name="s",
                                  num_cores=NC, num_subcores=NS)
    sc = pl.kernel(
        _sc_body,
        out_type=jax.ShapeDtypeStruct((NW, G * D), jnp.float32),
        mesh=mesh,
        compiler_params=pltpu.CompilerParams(needs_layout_passes=False),
        scratch_types=[
            pltpu.VMEM((C, D), jnp.float32),
            pltpu.VMEM((C,), jnp.int32),
            pltpu.VMEM((C, D), jnp.float32),
            pltpu.VMEM((C,), jnp.int32),
            pltpu.VMEM((G * D,), jnp.float32),
            pltpu.VMEM((D,), jnp.float32),
            pltpu.VMEM((L,), jnp.float32),
            pltpu.VMEM((L * L,), jnp.float32),
            pltpu.SemaphoreType.DMA,
            pltpu.SemaphoreType.DMA,
        ],
    )
    parts = sc(edge_feats, ids32, w_flat, b_pad)
    out = pl.pallas_call(
        _combine_body,
        out_shape=jax.ShapeDtypeStruct((G, D), jnp.float32),
    )(parts.reshape(NW, G, D))
    return out


def kernel(edge_feats, segment_ids, num_graphs, W, b):
    ids32 = segment_ids.astype(jnp.int32)
    w_flat = W.reshape(D)
    b_pad = jnp.broadcast_to(b.reshape(1), (L,)).astype(jnp.float32)
    return _run(edge_feats, ids32, w_flat, b_pad)


# speedup vs baseline: 5.3904x; 1.5608x over previous
import functools

import jax
import jax.numpy as jnp
from jax import lax
from jax.experimental import pallas as pl
from jax.experimental.pallas import tpu as pltpu
from jax.experimental.pallas import tpu_sc as plsc

E = 320000
D = 128
G = 256
NC = 2
NS = 16
L = 16
NW = NC * NS
EW = E // NW
C = 80
NCHUNK = EW // C
GROUPS = C // L


def _tanh(z):
    a = jnp.abs(z)
    e = jnp.exp(a * (-2.0))
    return jnp.sign(z) * (1.0 - e) / (1.0 + e)


def _sc_body(feats_hbm, ids_hbm, w_hbm, b_hbm, out_hbm,
             bufA, idsA, bufB, idsB, acc, wbuf, bbuf, smat, semA, semB):
    wid = lax.axis_index("s") * NC + lax.axis_index("c")
    row0 = wid * EW

    pltpu.sync_copy(w_hbm, wbuf)
    pltpu.sync_copy(b_hbm, bbuf)
    bv = bbuf[...]
    wv = [wbuf[pl.ds(j * L, L)] for j in range(D // L)]
    lanes = lax.iota(jnp.int32, L)

    def _zero(i, _):
        acc[pl.ds(i * L, L)] = jnp.zeros((L,), jnp.float32)
        return 0
    lax.fori_loop(0, G * D // L, _zero, 0)

    def _start(ci, buf, ids, sem):
        base = row0 + ci * C
        pltpu.async_copy(feats_hbm.at[pl.ds(base, C)], buf, sem)
        pltpu.async_copy(ids_hbm.at[pl.ds(base, C)], ids, sem)

    def _wait(buf, ids, sem):
        pltpu.make_async_copy(feats_hbm.at[pl.ds(0, C)], buf, sem).wait()
        pltpu.make_async_copy(ids_hbm.at[pl.ds(0, C)], ids, sem).wait()

    def _dg(x, perm):
        return x.at[perm].get(mode="promise_in_bounds")

    splat0 = jnp.zeros((L,), jnp.int32)
    splat1 = jnp.full((L,), 1, jnp.int32)
    p1, p2, p4, p8 = (lanes ^ 1, lanes ^ 2, lanes ^ 4, lanes ^ 8)
    m1 = (lanes & 1) == 0

    def _process(buf, ids_v):
        def _group(gi, _):
            r0 = gi * L
            idv = jnp.minimum(ids_v[pl.ds(r0, L)], G - 1)
            seg_first = idv[0]
            seg_last = idv[L - 1]

            def _uniform(_):
                gsum = [jnp.zeros((L,), jnp.float32) for _ in range(D // L)]
                for r in range(0, L, 2):
                    xa = [buf[r0 + r, pl.ds(j * L, L)] for j in range(D // L)]
                    xb = [buf[r0 + r + 1, pl.ds(j * L, L)]
                          for j in range(D // L)]
                    sa = xa[0] * wv[0]
                    sb = xb[0] * wv[0]
                    for j in range(1, D // L):
                        sa = sa + xa[j] * wv[j]
                        sb = sb + xb[j] * wv[j]
                    c = (jnp.where(m1, sa, sb)
                         + jnp.where(m1, _dg(sa, p1), _dg(sb, p1)))
                    t = c + _dg(c, p2)
                    t = t + _dg(t, p4)
                    t = t + _dg(t, p8)
                    wp = _tanh(t + bv)
                    wa = _dg(wp, splat0)
                    wb = _dg(wp, splat1)
                    for j in range(D // L):
                        gsum[j] = gsum[j] + xa[j] * wa + xb[j] * wb
                off = seg_first * D
                for j in range(D // L):
                    plsc.addupdate(acc.at[pl.ds(off + j * L, L)], gsum[j])
                return 0

            def _mixed(_):
                for r in range(L):
                    row = r0 + r
                    s = buf[row, pl.ds(0, L)] * wv[0]
                    for j in range(1, D // L):
                        s = s + buf[row, pl.ds(j * L, L)] * wv[j]
                    smat[pl.ds(r * L, L)] = s
                hv = plsc.load_gather(smat, [lanes * L])
                for j in range(1, L):
                    hv = hv + plsc.load_gather(smat, [lanes * L + j])
                wg = _tanh(hv + bv)
                for r in range(L):
                    rfull = jnp.full((L,), r, jnp.int32)
                    wsp = _dg(wg, rfull)
                    base_idx = _dg(idv, rfull) * D + lanes
                    for j in range(D // L):
                        x = buf[r0 + r, pl.ds(j * L, L)] * wsp
                        plsc.addupdate_scatter(acc, [base_idx + j * L], x)
                return 0

            lax.cond(seg_first == seg_last, _uniform, _mixed, 0)
            return 0

        lax.fori_loop(0, GROUPS, _group, 0)

    _start(0, bufA, idsA, semA)

    def _pair(p, _):
        ci = p * 2
        _wait(bufA, idsA, semA)
        _start(ci + 1, bufB, idsB, semB)
        _process(bufA, idsA)
        _wait(bufB, idsB, semB)
        _start(ci + 2, bufA, idsA, semA)
        _process(bufB, idsB)
        return 0

    lax.fori_loop(0, NCHUNK // 2, _pair, 0)
    _wait(bufA, idsA, semA)
    _process(bufA, idsA)
    pltpu.sync_copy(acc, out_hbm.at[wid])


def _combine_body(parts_ref, o_ref):
    o_ref[...] = jnp.sum(parts_ref[...], axis=0)


@jax.jit
def _run(edge_feats, ids32, w_flat, b_pad):
    mesh = plsc.VectorSubcoreMesh(core_axis_name="c", subcore_axis_name="s",
                                  num_cores=NC, num_subcores=NS)
    sc = pl.kernel(
        _sc_body,
        out_type=jax.ShapeDtypeStruct((NW, G * D), jnp.float32),
        mesh=mesh,
        compiler_params=pltpu.CompilerParams(needs_layout_passes=False),
        scratch_types=[
            pltpu.VMEM((C, D), jnp.float32),
            pltpu.VMEM((C,), jnp.int32),
            pltpu.VMEM((C, D), jnp.float32),
            pltpu.VMEM((C,), jnp.int32),
            pltpu.VMEM((G * D,), jnp.float32),
            pltpu.VMEM((D,), jnp.float32),
            pltpu.VMEM((L,), jnp.float32),
            pltpu.VMEM((L * L,), jnp.float32),
            pltpu.SemaphoreType.DMA,
            pltpu.SemaphoreType.DMA,
        ],
    )
    parts = sc(edge_feats, ids32, w_flat, b_pad)
    out = pl.pallas_call(
        _combine_body,
        out_shape=jax.ShapeDtypeStruct((G, D), jnp.float32),
    )(parts.reshape(NW, G, D))
    return out


def kernel(edge_feats, segment_ids, num_graphs, W, b):
    ids32 = segment_ids.astype(jnp.int32)
    w_flat = W.reshape(D)
    b_pad = jnp.broadcast_to(b.reshape(1), (L,)).astype(jnp.float32)
    return _run(edge_feats, ids32, w_flat, b_pad)
